# baseline (speedup 1.0000x reference)
import jax
import jax.numpy as jnp
from jax import lax
from jax.experimental import pallas as pl
from jax.experimental.pallas import tpu as pltpu
from jax.experimental.pallas import tpu_sc as plsc

_M = 1000000
_D = 16
_B = 16384

_NC = 2
_NS = 16
_NW = _NC * _NS

_RPT = _M // _NW
_SLAB = 3125
_NSLAB = _RPT // _SLAB

_BPW = _B // _NW
_CH = 128
_NCH = _BPW // _CH

_MESH = plsc.VectorSubcoreMesh(core_axis_name="c", subcore_axis_name="s")
_PARAMS = pltpu.CompilerParams(use_tc_tiling_on_sc=False)


def _dense_body(x_hbm, out_hbm, buf0, buf1, sem0, sem1, osem0, osem1):
    wid = lax.axis_index("s") * _NC + lax.axis_index("c")
    lo = wid * _RPT
    bufs = (buf0, buf1)
    gsems = (sem0, sem1)
    osems = (osem0, osem1)

    def dbl(buf):
        def body(i, _):
            buf[i, :] = buf[i, :] + buf[i, :]
            return 0

        lax.fori_loop(0, _SLAB, body, 0, unroll=8)

    gh = [None] * _NSLAB
    sh = [None] * _NSLAB
    gh[0] = pltpu.async_copy(
        x_hbm.at[pl.ds(lo, _SLAB), :], bufs[0], gsems[0]
    )
    for s in range(_NSLAB):
        b = s % 2
        gh[s].wait()
        if s > 0:
            sh[s - 1].wait()
        if s + 1 < _NSLAB:
            gh[s + 1] = pltpu.async_copy(
                x_hbm.at[pl.ds(lo + (s + 1) * _SLAB, _SLAB), :],
                bufs[(s + 1) % 2],
                gsems[(s + 1) % 2],
            )
        dbl(bufs[b])
        sh[s] = pltpu.async_copy(
            bufs[b], out_hbm.at[pl.ds(lo + s * _SLAB, _SLAB), :], osems[b]
        )
    sh[_NSLAB - 1].wait()


_sc_dense = pl.kernel(
    _dense_body,
    out_type=jax.ShapeDtypeStruct((_M, _D), jnp.float32),
    mesh=_MESH,
    compiler_params=_PARAMS,
    scratch_types=[
        pltpu.VMEM((_SLAB, _D), jnp.float32),
        pltpu.VMEM((_SLAB, _D), jnp.float32),
        pltpu.SemaphoreType.DMA,
        pltpu.SemaphoreType.DMA,
        pltpu.SemaphoreType.DMA,
        pltpu.SemaphoreType.DMA,
    ],
)


def _scatter_body(y_hbm, idx_hbm, out_ref, idx_v, rows_v, sem):
    wid = lax.axis_index("s") * _NC + lax.axis_index("c")
    pltpu.sync_copy(idx_hbm.at[pl.ds(wid * _NCH, _NCH)], idx_v)
    pltpu.sync_copy(y_hbm.at[pl.ds(wid * _BPW, _BPW)], rows_v)

    def body(i, _):
        rows_v[i, :] = rows_v[i, :] + rows_v[i, :]
        return 0

    lax.fori_loop(0, _BPW, body, 0, unroll=8)
    copies = [
        pltpu.async_copy(
            rows_v.at[pl.ds(j * _CH, _CH)], out_ref.at[idx_v.at[j]], sem
        )
        for j in range(_NCH)
    ]
    for c in copies:
        c.wait()


_sc_scatter = pl.kernel(
    _scatter_body,
    out_type=(),
    mesh=_MESH,
    compiler_params=_PARAMS,
    scratch_types=[
        pltpu.VMEM((_NCH, _CH), jnp.int32),
        pltpu.VMEM((_BPW, _D), jnp.float32),
        pltpu.SemaphoreType.DMA,
    ],
)


def kernel(x, y, index):
    xx = _sc_dense(x)
    idx2 = index.reshape(_NW * _NCH, _CH)
    out_ref = jax.new_ref(xx)
    _sc_scatter(y, idx2, out_ref)
    return jax.freeze(out_ref)


# speedup vs baseline: 1.7744x; 1.7744x over previous
import jax
import jax.numpy as jnp
from jax import lax
from jax.experimental import pallas as pl
from jax.experimental.pallas import tpu as pltpu
from jax.experimental.pallas import tpu_sc as plsc

_M = 1000000
_D = 16
_B = 16384

_NC = 2
_NS = 16
_NW = _NC * _NS

_W = 2048
_NFULL = 488
_SPECIAL = 488
_SPECIAL_W = 512
_SPECIAL_OWNER = _SPECIAL % _NW
_TAIL0 = 999936
_TAILN = _M - _TAIL0
_TAIL_OWNER = 9

_BPW = _B // _NW

_MESH = plsc.VectorSubcoreMesh(core_axis_name="c", subcore_axis_name="s")
_PARAMS = pltpu.CompilerParams(use_tc_tiling_on_sc=True, needs_layout_passes=False)
_IOTA = lambda: lax.iota(jnp.int32, 16)


def _pos_body(idx_hbm, pos_ref, idx_v, vals_v, sem):
    wid = lax.axis_index("s") * _NC + lax.axis_index("c")
    base = wid * _BPW
    pltpu.sync_copy(idx_hbm.at[pl.ds(wid * 4, 4)], idx_v)
    for j in range(4):
        for k in range(8):
            vals_v[j, pl.ds(k * 16, 16)] = base + j * 128 + k * 16 + _IOTA()
    copies = [
        pltpu.async_copy(vals_v.at[j], pos_ref.at[idx_v.at[j]], sem)
        for j in range(4)
    ]
    for c in copies:
        c.wait()


_sc_pos = pl.kernel(
    _pos_body,
    out_type=(),
    mesh=_MESH,
    compiler_params=pltpu.CompilerParams(use_tc_tiling_on_sc=False),
    scratch_types=[
        pltpu.VMEM((4, 128), jnp.int32),
        pltpu.VMEM((4, 128), jnp.int32),
        pltpu.SemaphoreType.DMA,
    ],
)


def _dense_body(xt_hbm, pos_hbm, yflat_hbm, xtail_hbm, out_hbm, tail_hbm,
                buf, pbuf, stage, tbuf, sem, psem, ysem):
    wid = lax.axis_index("s") * _NC + lax.axis_index("c")

    def process_slab(c0, ncols):
        gh = pltpu.async_copy(xt_hbm.at[:, pl.ds(c0, ncols)],
                              buf.at[:, pl.ds(0, ncols)], sem)
        ph = pltpu.async_copy(pos_hbm.at[pl.ds(c0, ncols)],
                              pbuf.at[pl.ds(0, ncols)], psem)
        gh.wait()

        def dbl(i, _):
            r = i // (ncols // 16)
            c = (i % (ncols // 16)) * 16
            buf[r, pl.ds(c, 16)] = buf[r, pl.ds(c, 16)] * 2.0
            return 0

        lax.fori_loop(0, _D * (ncols // 16), dbl, 0, unroll=8)
        ph.wait()

        def scan(ci, _):
            pv = pbuf[pl.ds(ci * 16, 16)]
            anyhit = jnp.max(pv) >= 0

            @pl.when(anyhit)
            def _():
                for k in range(16):
                    @pl.when(pv[k] >= 0)
                    def _():
                        p = pv[k]
                        ev = p * _D + _IOTA()
                        pltpu.async_copy(
                            yflat_hbm.at[ev], stage, ysem
                        ).wait()
                        v = stage[...]
                        plsc.store_scatter(
                            buf,
                            [_IOTA(), jnp.full((16,), ci * 16 + k, jnp.int32)],
                            v + v,
                        )
            return 0

        lax.fori_loop(0, ncols // 16, scan, 0)
        pltpu.async_copy(buf.at[:, pl.ds(0, ncols)],
                         out_hbm.at[:, pl.ds(c0, ncols)], sem).wait()

    nslabs = (_NFULL - wid + _NW - 1) // _NW

    def slab_loop(s, _):
        g = s * _NW + wid
        c0 = pl.multiple_of(g * _W, 128)
        process_slab(c0, _W)
        return 0

    lax.fori_loop(0, nslabs, slab_loop, 0)

    @pl.when(wid == _SPECIAL_OWNER)
    def _():
        process_slab(pl.multiple_of(_SPECIAL * _W, 128), _SPECIAL_W)

    @pl.when(wid == _TAIL_OWNER)
    def _():
        pltpu.sync_copy(xtail_hbm, tbuf)
        ph = pltpu.async_copy(pos_hbm.at[pl.ds(_TAIL0, _TAILN)],
                              pbuf.at[pl.ds(0, _TAILN)], psem)

        def dblt(i, _):
            tbuf[pl.ds(i * 16, 16)] = tbuf[pl.ds(i * 16, 16)] * 2.0
            return 0

        lax.fori_loop(0, _TAILN * _D // 16, dblt, 0, unroll=8)
        ph.wait()

        def scant(ci, _):
            pv = pbuf[pl.ds(ci * 16, 16)]

            @pl.when(jnp.max(pv) >= 0)
            def _():
                for k in range(16):
                    @pl.when(pv[k] >= 0)
                    def _():
                        p = pv[k]
                        ev = p * _D + _IOTA()
                        pltpu.async_copy(
                            yflat_hbm.at[ev], stage, ysem
                        ).wait()
                        v = stage[...]
                        tbuf[pl.ds((ci * 16 + k) * _D, _D)] = v + v
            return 0

        lax.fori_loop(0, _TAILN // 16, scant, 0)
        pltpu.sync_copy(tbuf, tail_hbm)


_sc_dense = pl.kernel(
    _dense_body,
    out_type=(
        jax.ShapeDtypeStruct((_D, _M), jnp.float32),
        jax.ShapeDtypeStruct((_TAILN * _D,), jnp.float32),
    ),
    mesh=_MESH,
    compiler_params=_PARAMS,
    scratch_types=[
        pltpu.VMEM((_D, _W), jnp.float32),
        pltpu.VMEM((_W,), jnp.int32),
        pltpu.VMEM((_D,), jnp.float32),
        pltpu.VMEM((_TAILN * _D,), jnp.float32),
        pltpu.SemaphoreType.DMA,
        pltpu.SemaphoreType.DMA,
        pltpu.SemaphoreType.DMA,
    ],
)


def kernel(x, y, index):
    xt = x.T
    yflat = y.reshape(_B * _D)
    xtail = x[_TAIL0:].reshape(_TAILN * _D)
    pos0 = jnp.full((_M,), -1, jnp.int32)
    pos_ref = jax.new_ref(pos0)
    _sc_pos(index.reshape(_NW * 4, 128), pos_ref)
    pos = jax.freeze(pos_ref)
    out_t, tail = _sc_dense(xt, pos, yflat, xtail)
    out = out_t.T
    return lax.dynamic_update_slice(out, tail.reshape(_TAILN, _D), (_TAIL0, 0))
